# Initial kernel scaffold; baseline (speedup 1.0000x reference)
import jax
import jax.numpy as jnp
from jax.experimental import pallas as pl


def kernel(user_emb, item_emb, user_intent, item_intent, edge_index):
    raise NotImplementedError("write your pallas kernel here")


# speedup vs baseline: 1.0094x; 1.0094x over previous
import functools

import jax
import jax.numpy as jnp
from jax.experimental import pallas as pl

NUM_USER = 25000
NUM_ITEM = 25000
N_NODES = NUM_USER + NUM_ITEM
DIM_E = 128
N_INTENTS = 128
N_LAYERS = 2

_ROW_BLK = 1000


def _intent_body(x_ref, w_ref, o_ref):
    x = x_ref[...]
    w = w_ref[...]
    logits = jnp.dot(x, w, preferred_element_type=jnp.float32)
    m = jnp.max(logits, axis=1, keepdims=True)
    e = jnp.exp(logits - m)
    p = e / jnp.sum(e, axis=1, keepdims=True)
    o_ref[...] = jnp.dot(p, w.T, preferred_element_type=jnp.float32)


def _intent(x, w):
    n = x.shape[0]
    grid = (n // _ROW_BLK,)
    return pl.pallas_call(
        _intent_body,
        grid=grid,
        in_specs=[
            pl.BlockSpec((_ROW_BLK, DIM_E), lambda i: (i, 0)),
            pl.BlockSpec((DIM_E, N_INTENTS), lambda i: (0, 0)),
        ],
        out_specs=pl.BlockSpec((_ROW_BLK, N_INTENTS), lambda i: (i, 0)),
        out_shape=jax.ShapeDtypeStruct((n, DIM_E), jnp.float32),
    )(x, w)


def _l2norm(x):
    n = jnp.linalg.norm(x, axis=1, keepdims=True)
    return x / jnp.maximum(n, 1e-12)


def kernel(user_emb, item_emb, user_intent, item_intent, edge_index):
    h = edge_index[:, 0].astype(jnp.int32)
    t = edge_index[:, 1].astype(jnp.int32)
    rows = jnp.concatenate([h, t])
    cols = jnp.concatenate([t, h])
    deg = jnp.zeros((N_NODES,), jnp.float32).at[rows].add(1.0) + 1e-07
    dinv = jnp.power(deg, -0.5)
    vals = dinv[rows] * dinv[cols]

    emb0 = jnp.concatenate([user_emb, item_emb], axis=0)
    cur = emb0
    gnn_list, int_list, gaa_list, iaa_list = [], [], [], []
    for _ in range(N_LAYERS):
        gnn = jnp.zeros((N_NODES, DIM_E), jnp.float32).at[rows].add(
            vals[:, None] * cur[cols])
        u_int = _intent(cur[:NUM_USER], user_intent)
        i_int = _intent(cur[NUM_USER:], item_intent)
        intl = jnp.concatenate([u_int, i_int], axis=0)
        alpha_g = (jnp.sum(_l2norm(gnn[h]) * _l2norm(gnn[t]), axis=1) + 1.0) / 2.0
        alpha_i = (jnp.sum(_l2norm(intl[h]) * _l2norm(intl[t]), axis=1) + 1.0) / 2.0
        cur_t = cur[t]
        gaa = jnp.zeros((N_NODES, DIM_E), jnp.float32).at[h].add(
            alpha_g[:, None] * cur_t)
        iaa = jnp.zeros((N_NODES, DIM_E), jnp.float32).at[h].add(
            alpha_i[:, None] * cur_t)
        gnn_list.append(gnn)
        int_list.append(intl)
        gaa_list.append(gaa)
        iaa_list.append(iaa)
        cur = gnn + intl + gaa + iaa + cur
    return jnp.stack([jnp.stack(gnn_list), jnp.stack(int_list),
                      jnp.stack(gaa_list), jnp.stack(iaa_list)])


# speedup vs baseline: 3.5817x; 3.5485x over previous
import jax
import jax.numpy as jnp
from jax import lax
from jax.experimental import pallas as pl
from jax.experimental.pallas import tpu as pltpu
from jax.experimental.pallas import tpu_sc as plsc

NUM_USER = 25000
NUM_ITEM = 25000
N_NODES = NUM_USER + NUM_ITEM
DIM_E = 128
N_LAYERS = 2
N_EDGES = 400000

NC = 2
NS = 16
NW = NC * NS
L = 16

NB = 8
BKT = 6250
ACC_R = 6272
RPT_S = ACC_R // NS

NA = 8
BKA = 3125
ACC_A = 3200
RPT_A = ACC_A // NS

EU_PAD = 401408
D_PAD = 2 * EU_PAD
EPT = D_PAD // NW
UPT = EU_PAD // NW
CAP = 4096
CAP_PAD = CAP + 16
CAPA = 2048
CAPA_PAD = CAPA + 16
KE = 512
KE2 = 448
KG = 128
KGA = 32
SENT = N_NODES
DEGW = 50176
DEGS = DEGW // NS

_f32 = jnp.float32
_i32 = jnp.int32


def _mesh():
    return plsc.VectorSubcoreMesh(core_axis_name="c", subcore_axis_name="s")


def _fill_zbuf(zbuf, rows):
    zf = jnp.zeros((L,), _f32)

    def zfill(i, carry):
        for j in range(DIM_E // L):
            zbuf[i, pl.ds(j * L, L)] = zf
        return carry
    lax.fori_loop(0, rows, zfill, 0)


def _zero_slice(zbuf, zr, acc, s, rpt):
    base = s * rpt
    for r in range(rpt // zr):
        pltpu.sync_copy(zbuf, acc.at[pl.ds(base + r * zr, zr)])


def _prep1_body(dst_hbm, src_hbm, frag_rel, frag_src, cnts_hbm, deg_hbm,
                dstc, srcc, degz, cntv, didx, onesb, cnt_s, deg_sh, *fb):
    fb_rel = fb[:NB]
    fb_src = fb[NB:]
    c = lax.axis_index("c")
    s = lax.axis_index("s")
    w = s * NC + c

    zf = jnp.zeros((L,), _f32)
    zi = jnp.zeros((L,), _i32)
    trash = jnp.full((L,), BKT, _i32)
    onev = jnp.ones((L,), _f32)

    @pl.when(s == 0)
    def _zero_deg():
        def zdeg(i, carry):
            degz[pl.ds(i * L, L)] = zf
            return carry
        lax.fori_loop(0, DEGW // L, zdeg, 0)
        pltpu.sync_copy(degz, deg_sh)
    plsc.subcore_barrier()

    for j in range(KG // L):
        onesb[pl.ds(j * L, L)] = onev

    def zfrag(i, carry):
        for bb in range(NB):
            fb_rel[bb][pl.ds(i * L, L)] = trash
            fb_src[bb][pl.ds(i * L, L)] = zi
        return carry
    lax.fori_loop(0, CAP_PAD // L, zfrag, 0)

    for bb in range(NB):
        cnt_s[bb] = 0

    def chunk(kc, carry):
        off = (kc * NW + w) * KE
        pltpu.sync_copy(dst_hbm.at[pl.ds(off, KE)], dstc)
        pltpu.sync_copy(src_hbm.at[pl.ds(off, KE)], srcc)

        for q in range(KE // KG):
            pltpu.sync_copy(dst_hbm.at[pl.ds(off + q * KG, KG)], didx)
            pltpu.sync_copy(onesb, deg_sh.at[didx], add=True)

        def vreg(j, carry2):
            d = dstc[pl.ds(j * L, L)]
            sv = srcc[pl.ds(j * L, L)]
            b = d // BKT
            rel = d - b * BKT
            for bb in range(NB):
                m = b == bb
                cnt = cnt_s[bb]
                plsc.store_compressed(fb_rel[bb].at[pl.ds(cnt, L)], rel, mask=m)
                plsc.store_compressed(fb_src[bb].at[pl.ds(cnt, L)], sv, mask=m)
                cnt_s[bb] = cnt + plsc.all_reduce_population_count(m)[0]
            return carry2
        lax.fori_loop(0, KE // L, vreg, 0)
        return carry
    lax.fori_loop(0, EPT // KE, chunk, 0)

    lanes = lax.broadcasted_iota(_i32, (L,), 0)
    for bb in range(NB):
        plsc.store_scatter(cntv, [lanes],
                           jnp.full((L,), cnt_s[bb], _i32), mask=lanes == bb)
        pltpu.sync_copy(fb_rel[bb].at[pl.ds(0, CAP)], frag_rel.at[bb, w])
        pltpu.sync_copy(fb_src[bb].at[pl.ds(0, CAP)], frag_src.at[bb, w])
    pltpu.sync_copy(cntv, cnts_hbm.at[w])
    plsc.subcore_barrier()

    @pl.when(s == 0)
    def _copy_deg():
        pltpu.sync_copy(deg_sh, deg_hbm.at[c])


def _prep1(dst_dir, src_dir):
    return pl.kernel(
        _prep1_body,
        out_type=(
            jax.ShapeDtypeStruct((NB, NW, CAP), _i32),
            jax.ShapeDtypeStruct((NB, NW, CAP), _i32),
            jax.ShapeDtypeStruct((NW, L), _i32),
            jax.ShapeDtypeStruct((NC, DEGW), _f32),
        ),
        mesh=_mesh(),
        compiler_params=pltpu.CompilerParams(needs_layout_passes=False),
        scratch_types=[
            pltpu.VMEM((KE,), _i32),
            pltpu.VMEM((KE,), _i32),
            pltpu.VMEM((DEGW,), _f32),
            pltpu.VMEM((L,), _i32),
            pltpu.VMEM((KG,), _i32),
            pltpu.VMEM((KG,), _f32),
            pltpu.SMEM((NB,), _i32),
            pltpu.VMEM_SHARED((DEGW,), _f32),
        ] + [pltpu.VMEM((CAP_PAD,), _i32) for _ in range(2 * NB)],
    )(dst_dir, src_dir)


def _prep2_body(h_hbm, t_hbm, frag_rel, frag_abs, frag_src, cnts_hbm,
                hc, tc, cntv, cnt_s, *fb):
    fb_rel = fb[:NA]
    fb_abs = fb[NA:2 * NA]
    fb_src = fb[2 * NA:]
    c = lax.axis_index("c")
    s = lax.axis_index("s")
    w = s * NC + c

    zi = jnp.zeros((L,), _i32)
    trash = jnp.full((L,), BKA, _i32)

    def zfrag(i, carry):
        for bb in range(NA):
            fb_rel[bb][pl.ds(i * L, L)] = trash
            fb_abs[bb][pl.ds(i * L, L)] = zi
            fb_src[bb][pl.ds(i * L, L)] = zi
        return carry
    lax.fori_loop(0, CAPA_PAD // L, zfrag, 0)

    for bb in range(NA):
        cnt_s[bb] = 0

    base = w * UPT

    def chunk(kc, carry):
        pltpu.sync_copy(h_hbm.at[pl.ds(base + kc * KE2, KE2)], hc)
        pltpu.sync_copy(t_hbm.at[pl.ds(base + kc * KE2, KE2)], tc)

        def vreg(j, carry2):
            hv = hc[pl.ds(j * L, L)]
            tv = tc[pl.ds(j * L, L)]
            b = hv // BKA
            rel = hv - b * BKA
            for bb in range(NA):
                m = b == bb
                cnt = cnt_s[bb]
                plsc.store_compressed(fb_rel[bb].at[pl.ds(cnt, L)], rel, mask=m)
                plsc.store_compressed(fb_abs[bb].at[pl.ds(cnt, L)], hv, mask=m)
                plsc.store_compressed(fb_src[bb].at[pl.ds(cnt, L)], tv, mask=m)
                cnt_s[bb] = cnt + plsc.all_reduce_population_count(m)[0]
            return carry2
        lax.fori_loop(0, KE2 // L, vreg, 0)
        return carry
    lax.fori_loop(0, UPT // KE2, chunk, 0)

    lanes = lax.broadcasted_iota(_i32, (L,), 0)
    for bb in range(NA):
        plsc.store_scatter(cntv, [lanes],
                           jnp.full((L,), cnt_s[bb], _i32), mask=lanes == bb)
        pltpu.sync_copy(fb_rel[bb].at[pl.ds(0, CAPA)], frag_rel.at[bb, w])
        pltpu.sync_copy(fb_abs[bb].at[pl.ds(0, CAPA)], frag_abs.at[bb, w])
        pltpu.sync_copy(fb_src[bb].at[pl.ds(0, CAPA)], frag_src.at[bb, w])
    pltpu.sync_copy(cntv, cnts_hbm.at[w])


def _prep2(hp, tp):
    return pl.kernel(
        _prep2_body,
        out_type=(
            jax.ShapeDtypeStruct((NA, NW, CAPA), _i32),
            jax.ShapeDtypeStruct((NA, NW, CAPA), _i32),
            jax.ShapeDtypeStruct((NA, NW, CAPA), _i32),
            jax.ShapeDtypeStruct((NW, L), _i32),
        ),
        mesh=_mesh(),
        compiler_params=pltpu.CompilerParams(needs_layout_passes=False),
        scratch_types=[
            pltpu.VMEM((KE2,), _i32),
            pltpu.VMEM((KE2,), _i32),
            pltpu.VMEM((L,), _i32),
            pltpu.SMEM((NA,), _i32),
        ] + [pltpu.VMEM((CAPA_PAD,), _i32) for _ in range(3 * NA)],
    )(hp, tp)


ZR_S = 56


def _spmm_body(curs_hbm, frag_rel, frag_src, cnts_hbm, out_hbm,
               cntv, sidx, ridx, rows, zbuf, acc):
    c = lax.axis_index("c")
    s = lax.axis_index("s")
    _fill_zbuf(zbuf, ZR_S)

    for b in range(NB // NC):
        bkt = (NB // NC) * c + b
        _zero_slice(zbuf, ZR_S, acc, s, RPT_S)
        plsc.subcore_barrier()
        for f in range(2):
            w = 2 * s + f
            pltpu.sync_copy(cnts_hbm.at[w], cntv)
            cnt = plsc.load_gather(cntv, [jnp.full((L,), bkt, _i32)])[0]
            nch = (cnt + KG - 1) // KG

            def chunk(k, carry):
                pltpu.sync_copy(frag_src.at[bkt, w, pl.ds(k * KG, KG)], sidx)
                pltpu.sync_copy(frag_rel.at[bkt, w, pl.ds(k * KG, KG)], ridx)
                pltpu.sync_copy(curs_hbm.at[sidx], rows)
                pltpu.sync_copy(rows, acc.at[ridx], add=True)
                return carry
            lax.fori_loop(0, nch, chunk, 0)
        plsc.subcore_barrier()
        pltpu.sync_copy(
            acc.at[pl.ds(s * RPT_S, RPT_S)],
            out_hbm.at[pl.ds(bkt * ACC_R + s * RPT_S, RPT_S)])
        plsc.subcore_barrier()


def _spmm(curs, frag_rel, frag_src, cnts):
    return pl.kernel(
        _spmm_body,
        out_type=jax.ShapeDtypeStruct((NB * ACC_R, DIM_E), _f32),
        mesh=_mesh(),
        compiler_params=pltpu.CompilerParams(needs_layout_passes=False),
        scratch_types=[
            pltpu.VMEM((L,), _i32),
            pltpu.VMEM((KG,), _i32),
            pltpu.VMEM((KG,), _i32),
            pltpu.VMEM((KG, DIM_E), _f32),
            pltpu.VMEM((ZR_S, DIM_E), _f32),
            pltpu.VMEM_SHARED((ACC_R, DIM_E), _f32),
        ],
    )(curs, frag_rel, frag_src, cnts)


ZR_A = 40


def _attn_body(ghat_hbm, ihat_hbm, cur_hbm, frag_rel, frag_abs, frag_src,
               cnts_hbm, gout_hbm, iout_hbm,
               cntv, ridx, hidx, tidx,
               ghv, gtv, ihv, itv, ctv, gbuf, ibuf, zbuf, accg, acci):
    c = lax.axis_index("c")
    s = lax.axis_index("s")
    zf = jnp.zeros((L,), _f32)
    _fill_zbuf(zbuf, ZR_A)

    for b in range(NA // NC):
        hb = (NA // NC) * c + b
        _zero_slice(zbuf, ZR_A, accg, s, RPT_A)
        _zero_slice(zbuf, ZR_A, acci, s, RPT_A)
        plsc.subcore_barrier()
        for f in range(2):
            w = 2 * s + f
            pltpu.sync_copy(cnts_hbm.at[w], cntv)
            cnt = plsc.load_gather(cntv, [jnp.full((L,), hb, _i32)])[0]
            nch = (cnt + KGA - 1) // KGA

            def chunk(k, carry):
                pltpu.sync_copy(frag_rel.at[hb, w, pl.ds(k * KGA, KGA)], ridx)
                pltpu.sync_copy(frag_abs.at[hb, w, pl.ds(k * KGA, KGA)], hidx)
                pltpu.sync_copy(frag_src.at[hb, w, pl.ds(k * KGA, KGA)], tidx)
                pltpu.sync_copy(ghat_hbm.at[hidx], ghv)
                pltpu.sync_copy(ghat_hbm.at[tidx], gtv)
                pltpu.sync_copy(ihat_hbm.at[hidx], ihv)
                pltpu.sync_copy(ihat_hbm.at[tidx], itv)
                pltpu.sync_copy(cur_hbm.at[tidx], ctv)

                def edge(e, carry2):
                    dg = zf
                    di = zf
                    for j in range(DIM_E // L):
                        sl = pl.ds(j * L, L)
                        dg = dg + ghv[e, sl] * gtv[e, sl]
                        di = di + ihv[e, sl] * itv[e, sl]
                    ag = (jnp.sum(dg) + 1.0) * 0.5
                    ai = (jnp.sum(di) + 1.0) * 0.5
                    agv = jnp.full((L,), ag, _f32)
                    aiv = jnp.full((L,), ai, _f32)
                    for j in range(DIM_E // L):
                        sl = pl.ds(j * L, L)
                        cr = ctv[e, sl]
                        gbuf[e, sl] = cr * agv
                        ibuf[e, sl] = cr * aiv
                    return carry2
                lax.fori_loop(0, KGA, edge, 0)
                pltpu.sync_copy(gbuf, accg.at[ridx], add=True)
                pltpu.sync_copy(ibuf, acci.at[ridx], add=True)
                return carry
            lax.fori_loop(0, nch, chunk, 0)
        plsc.subcore_barrier()
        off = hb * ACC_A + s * RPT_A
        pltpu.sync_copy(accg.at[pl.ds(s * RPT_A, RPT_A)],
                        gout_hbm.at[pl.ds(off, RPT_A)])
        pltpu.sync_copy(acci.at[pl.ds(s * RPT_A, RPT_A)],
                        iout_hbm.at[pl.ds(off, RPT_A)])
        plsc.subcore_barrier()


def _attn(ghat, ihat, cur, frag_rel, frag_abs, frag_src, cnts):
    return pl.kernel(
        _attn_body,
        out_type=(
            jax.ShapeDtypeStruct((NA * ACC_A, DIM_E), _f32),
            jax.ShapeDtypeStruct((NA * ACC_A, DIM_E), _f32),
        ),
        mesh=_mesh(),
        compiler_params=pltpu.CompilerParams(needs_layout_passes=False),
        scratch_types=[
            pltpu.VMEM((L,), _i32),
            pltpu.VMEM((KGA,), _i32),
            pltpu.VMEM((KGA,), _i32),
            pltpu.VMEM((KGA,), _i32),
            pltpu.VMEM((KGA, DIM_E), _f32),
            pltpu.VMEM((KGA, DIM_E), _f32),
            pltpu.VMEM((KGA, DIM_E), _f32),
            pltpu.VMEM((KGA, DIM_E), _f32),
            pltpu.VMEM((KGA, DIM_E), _f32),
            pltpu.VMEM((KGA, DIM_E), _f32),
            pltpu.VMEM((KGA, DIM_E), _f32),
            pltpu.VMEM((ZR_A, DIM_E), _f32),
            pltpu.VMEM_SHARED((ACC_A, DIM_E), _f32),
            pltpu.VMEM_SHARED((ACC_A, DIM_E), _f32),
        ],
    )(ghat, ihat, cur, frag_rel, frag_abs, frag_src, cnts)


_ROW_BLK = 1000


def _intent_body(x_ref, w_ref, o_int, o_hat):
    x = x_ref[...]
    w = w_ref[...]
    logits = jnp.dot(x, w, preferred_element_type=_f32)
    m = jnp.max(logits, axis=1, keepdims=True)
    e = jnp.exp(logits - m)
    p = e / jnp.sum(e, axis=1, keepdims=True)
    intl = jnp.dot(p, w.T, preferred_element_type=_f32)
    o_int[...] = intl
    n = jnp.sqrt(jnp.sum(intl * intl, axis=1, keepdims=True))
    o_hat[...] = intl / jnp.maximum(n, 1e-12)


def _intent(x, w):
    n = x.shape[0]
    return pl.pallas_call(
        _intent_body,
        grid=(n // _ROW_BLK,),
        in_specs=[
            pl.BlockSpec((_ROW_BLK, DIM_E), lambda i: (i, 0)),
            pl.BlockSpec((DIM_E, DIM_E), lambda i: (0, 0)),
        ],
        out_specs=[
            pl.BlockSpec((_ROW_BLK, DIM_E), lambda i: (i, 0)),
            pl.BlockSpec((_ROW_BLK, DIM_E), lambda i: (i, 0)),
        ],
        out_shape=[
            jax.ShapeDtypeStruct((n, DIM_E), _f32),
            jax.ShapeDtypeStruct((n, DIM_E), _f32),
        ],
    )(x, w)


def _fin_body(raw_ref, dinv_ref, o_gnn, o_hat):
    raw = raw_ref[...]
    dv = dinv_ref[...]
    gnn = raw * dv
    o_gnn[...] = gnn
    n = jnp.sqrt(jnp.sum(gnn * gnn, axis=1, keepdims=True))
    o_hat[...] = gnn / jnp.maximum(n, 1e-12)


def _finalize(raw, dinv2d):
    return pl.pallas_call(
        _fin_body,
        grid=(N_NODES // _ROW_BLK,),
        in_specs=[
            pl.BlockSpec((_ROW_BLK, DIM_E), lambda i: (i, 0)),
            pl.BlockSpec((_ROW_BLK, 1), lambda i: (i, 0)),
        ],
        out_specs=[
            pl.BlockSpec((_ROW_BLK, DIM_E), lambda i: (i, 0)),
            pl.BlockSpec((_ROW_BLK, DIM_E), lambda i: (i, 0)),
        ],
        out_shape=[
            jax.ShapeDtypeStruct((N_NODES, DIM_E), _f32),
            jax.ShapeDtypeStruct((N_NODES, DIM_E), _f32),
        ],
    )(raw, dinv2d)


def kernel(user_emb, item_emb, user_intent, item_intent, edge_index):
    h = edge_index[:, 0].astype(_i32)
    t = edge_index[:, 1].astype(_i32)
    pad = jnp.full((EU_PAD - N_EDGES,), SENT, _i32)
    hp = jnp.concatenate([h, pad])
    tp = jnp.concatenate([t, pad])
    dst_dir = jnp.concatenate([hp, tp])
    src_dir = jnp.concatenate([tp, hp])

    frag_rel, frag_src, cnts, deg_parts = _prep1(dst_dir, src_dir)
    afrag_rel, afrag_abs, afrag_src, acnts = _prep2(hp, tp)
    deg = deg_parts.sum(axis=0)[:N_NODES] + 1e-07
    dinv2d = jnp.power(deg, -0.5)[:, None]

    cur = jnp.concatenate([user_emb, item_emb], axis=0)
    gnn_list, int_list, gaa_list, iaa_list = [], [], [], []
    for _ in range(N_LAYERS):
        curs = cur * dinv2d
        raw = _spmm(curs, frag_rel, frag_src, cnts)
        raw = raw.reshape(NB, ACC_R, DIM_E)[:, :BKT].reshape(N_NODES, DIM_E)
        gnn, ghat = _finalize(raw, dinv2d)
        ui, uh = _intent(cur[:NUM_USER], user_intent)
        ii, ih = _intent(cur[NUM_USER:], item_intent)
        intl = jnp.concatenate([ui, ii], axis=0)
        ihat = jnp.concatenate([uh, ih], axis=0)
        graw, iraw = _attn(ghat, ihat, cur, afrag_rel, afrag_abs,
                           afrag_src, acnts)
        gaa_u = graw.reshape(NA, ACC_A, DIM_E)[:, :BKA].reshape(NUM_USER, DIM_E)
        iaa_u = iraw.reshape(NA, ACC_A, DIM_E)[:, :BKA].reshape(NUM_USER, DIM_E)
        zeros_i = jnp.zeros((NUM_ITEM, DIM_E), _f32)
        gaa = jnp.concatenate([gaa_u, zeros_i], axis=0)
        iaa = jnp.concatenate([iaa_u, zeros_i], axis=0)
        gnn_list.append(gnn)
        int_list.append(intl)
        gaa_list.append(gaa)
        iaa_list.append(iaa)
        cur = gnn + intl + gaa + iaa + cur
    return jnp.stack([jnp.stack(gnn_list), jnp.stack(int_list),
                      jnp.stack(gaa_list), jnp.stack(iaa_list)])
